# initial kernel scaffold (unmeasured)
import jax
import jax.numpy as jnp
from jax import lax
from jax.experimental import pallas as pl
from jax.experimental.pallas import tpu as pltpu

N_DEV = 32


def kernel(x, w_mat):
    m_per, k = x.shape
    _, n_per = w_mat.shape

    xb = x.astype(jnp.bfloat16)
    wb = w_mat.astype(jnp.bfloat16)

    def body(x_ref, w_ref, out_ref, comm_ref, amax_ref,
             ring_send_sems, ring_recv_sems, a_send_sems, a_recv_sems):
        my = lax.axis_index("i")
        left = lax.rem(my + (N_DEV - 1), N_DEV)
        right = lax.rem(my + 1, N_DEV)

        barrier = pltpu.get_barrier_semaphore()
        pl.semaphore_signal(barrier, inc=1, device_id=(left,),
                            device_id_type=pl.DeviceIdType.MESH)
        pl.semaphore_signal(barrier, inc=1, device_id=(right,),
                            device_id_type=pl.DeviceIdType.MESH)
        pl.semaphore_wait(barrier, 2)

        ring_rdmas = []
        for h in range(N_DEV - 1):
            o_send = lax.rem(my + (N_DEV - h), N_DEV)
            src = x_ref if h == 0 else comm_ref.at[o_send]
            rdma = pltpu.make_async_remote_copy(
                src_ref=src,
                dst_ref=comm_ref.at[o_send],
                send_sem=ring_send_sems.at[h],
                recv_sem=ring_recv_sems.at[h],
                device_id=(right,),
                device_id_type=pl.DeviceIdType.MESH,
            )
            rdma.start()
            ring_rdmas.append(rdma)

            src_blk = x_ref[...] if h == 0 else comm_ref[o_send]
            blk = jnp.dot(src_blk, w_ref[...],
                          preferred_element_type=jnp.float32)
            out_ref[pl.ds(o_send * m_per, m_per), :] = jnp.maximum(blk, 0.0)

            rdma.wait_recv()

        o_last = lax.rem(my + 1, N_DEV)
        blk = jnp.dot(comm_ref[o_last], w_ref[...],
                      preferred_element_type=jnp.float32)
        out_ref[pl.ds(o_last * m_per, m_per), :] = jnp.maximum(blk, 0.0)

        local_max = jnp.max(out_ref[...])
        amax_ref[my] = jnp.full((8, 128), local_max, jnp.float32)
        a_rdmas = []
        for d in range(1, N_DEV):
            tgt = lax.rem(my + d, N_DEV)
            r = pltpu.make_async_remote_copy(
                src_ref=amax_ref.at[my],
                dst_ref=amax_ref.at[my],
                send_sem=a_send_sems.at[d - 1],
                recv_sem=a_recv_sems.at[d - 1],
                device_id=(tgt,),
                device_id_type=pl.DeviceIdType.MESH,
            )
            r.start()
            a_rdmas.append(r)
        for r in a_rdmas:
            r.wait()

        gmax = jnp.max(amax_ref[...])
        scale = gmax / 127.0
        y = out_ref[...]
        q = jnp.clip(jnp.round(y / scale), 0.0, 127.0)
        out_ref[...] = q * scale

        for rdma in ring_rdmas:
            rdma.wait_send()

    return pl.pallas_call(
        body,
        out_shape=jax.ShapeDtypeStruct((N_DEV * m_per, n_per), jnp.float32),
        in_specs=[
            pl.BlockSpec(memory_space=pltpu.VMEM),
            pl.BlockSpec(memory_space=pltpu.VMEM),
        ],
        out_specs=pl.BlockSpec(memory_space=pltpu.VMEM),
        scratch_shapes=[
            pltpu.VMEM((N_DEV, m_per, k), jnp.bfloat16),
            pltpu.VMEM((N_DEV, 8, 128), jnp.float32),
            pltpu.SemaphoreType.DMA((N_DEV - 1,)),
            pltpu.SemaphoreType.DMA((N_DEV - 1,)),
            pltpu.SemaphoreType.DMA((N_DEV - 1,)),
            pltpu.SemaphoreType.DMA((N_DEV - 1,)),
        ],
        compiler_params=pltpu.CompilerParams(collective_id=0),
    )(xb, wb)


# baseline (device time: 429245 ns/iter reference)
import jax
import jax.numpy as jnp
from jax import lax
from jax.experimental import pallas as pl
from jax.experimental.pallas import tpu as pltpu

N_DEV = 32


def kernel(x, w_mat):
    m_per, k = x.shape
    _, n_per = w_mat.shape

    xb = x.astype(jnp.bfloat16)
    wb = w_mat.astype(jnp.bfloat16)

    def body(x_ref, w_ref, out_ref, comm_ref, amax_ref,
             ring_send_sems, ring_recv_sems, a_send_sems, a_recv_sems):
        my = lax.axis_index("i")
        left = lax.rem(my + (N_DEV - 1), N_DEV)
        right = lax.rem(my + 1, N_DEV)

        barrier = pltpu.get_barrier_semaphore()
        pl.semaphore_signal(barrier, inc=1, device_id=(left,),
                            device_id_type=pl.DeviceIdType.MESH)
        pl.semaphore_signal(barrier, inc=1, device_id=(right,),
                            device_id_type=pl.DeviceIdType.MESH)
        pl.semaphore_wait(barrier, 2)

        ring_rdmas = []
        for h in range(N_DEV - 1):
            o_send = lax.rem(my + (N_DEV - h), N_DEV)
            src = x_ref if h == 0 else comm_ref.at[o_send]
            rdma = pltpu.make_async_remote_copy(
                src_ref=src,
                dst_ref=comm_ref.at[o_send],
                send_sem=ring_send_sems.at[h],
                recv_sem=ring_recv_sems.at[h],
                device_id=(right,),
                device_id_type=pl.DeviceIdType.MESH,
            )
            rdma.start()
            ring_rdmas.append(rdma)

            src_blk = x_ref[...] if h == 0 else comm_ref[o_send]
            blk = jnp.dot(src_blk, w_ref[...],
                          preferred_element_type=jnp.float32)
            out_ref[pl.ds(o_send * m_per, m_per), :] = jnp.maximum(blk, 0.0)

            rdma.wait_recv()

        o_last = lax.rem(my + 1, N_DEV)
        blk = jnp.dot(comm_ref[o_last], w_ref[...],
                      preferred_element_type=jnp.float32)
        out_ref[pl.ds(o_last * m_per, m_per), :] = jnp.maximum(blk, 0.0)

        local_max = jnp.max(out_ref[...])
        amax_ref[my] = jnp.full((8, 128), local_max, jnp.float32)
        a_rdmas = []
        for d in range(1, N_DEV):
            tgt = lax.rem(my + d, N_DEV)
            r = pltpu.make_async_remote_copy(
                src_ref=amax_ref.at[my],
                dst_ref=amax_ref.at[my],
                send_sem=a_send_sems.at[d - 1],
                recv_sem=a_recv_sems.at[d - 1],
                device_id=(tgt,),
                device_id_type=pl.DeviceIdType.MESH,
            )
            r.start()
            a_rdmas.append(r)
        for r in a_rdmas:
            r.wait()

        gmax = jnp.max(amax_ref[...])
        scale = gmax / 127.0
        y = out_ref[...]
        q = jnp.clip(jnp.round(y / scale), 0.0, 127.0)
        out_ref[...] = q * scale

        for rdma in ring_rdmas:
            rdma.wait_send()

    return pl.pallas_call(
        body,
        out_shape=jax.ShapeDtypeStruct((N_DEV * m_per, n_per), jnp.float32),
        in_specs=[
            pl.BlockSpec(memory_space=pltpu.VMEM),
            pl.BlockSpec(memory_space=pltpu.VMEM),
        ],
        out_specs=pl.BlockSpec(memory_space=pltpu.VMEM),
        scratch_shapes=[
            pltpu.VMEM((N_DEV, m_per, k), jnp.bfloat16),
            pltpu.VMEM((N_DEV, 8, 128), jnp.float32),
            pltpu.SemaphoreType.DMA((N_DEV - 1,)),
            pltpu.SemaphoreType.DMA((N_DEV - 1,)),
            pltpu.SemaphoreType.DMA((N_DEV - 1,)),
            pltpu.SemaphoreType.DMA((N_DEV - 1,)),
        ],
        compiler_params=pltpu.CompilerParams(
            collective_id=0,
            vmem_limit_bytes=64 * 1024 * 1024,
        ),
    )(xb, wb)


# device time: 415779 ns/iter; 1.0324x vs baseline; 1.0324x over previous
import jax
import jax.numpy as jnp
from jax import lax
from jax.experimental import pallas as pl
from jax.experimental.pallas import tpu as pltpu

N_DEV = 32


def kernel(x, w_mat):
    m_per, k = x.shape
    _, n_per = w_mat.shape

    xb = x.astype(jnp.bfloat16)
    wb = w_mat.astype(jnp.bfloat16)

    n_cw = N_DEV // 2
    n_ccw = N_DEV - 1 - n_cw

    def body(x_ref, w_ref, out_ref, comm_ref, amax_ref,
             cw_send_sems, cw_recv_sems, ccw_send_sems, ccw_recv_sems,
             a_send_sems, a_recv_sems):
        my = lax.axis_index("i")
        left = lax.rem(my + (N_DEV - 1), N_DEV)
        right = lax.rem(my + 1, N_DEV)

        barrier = pltpu.get_barrier_semaphore()
        pl.semaphore_signal(barrier, inc=1, device_id=(left,),
                            device_id_type=pl.DeviceIdType.MESH)
        pl.semaphore_signal(barrier, inc=1, device_id=(right,),
                            device_id_type=pl.DeviceIdType.MESH)
        pl.semaphore_wait(barrier, 2)

        def gemm(origin):
            blk = jnp.dot(comm_ref[origin], w_ref[...],
                          preferred_element_type=jnp.float32)
            out_ref[pl.ds(origin * m_per, m_per), :] = jnp.maximum(blk, 0.0)

        ring_rdmas = []
        for s in range(n_cw):
            if s < n_cw:
                o_cw = lax.rem(my + (N_DEV - s), N_DEV)
                cw = pltpu.make_async_remote_copy(
                    src_ref=x_ref if s == 0 else comm_ref.at[o_cw],
                    dst_ref=comm_ref.at[o_cw],
                    send_sem=cw_send_sems.at[s],
                    recv_sem=cw_recv_sems.at[s],
                    device_id=(right,),
                    device_id_type=pl.DeviceIdType.MESH,
                )
                cw.start()
                ring_rdmas.append(cw)
            if s < n_ccw:
                o_ccw = lax.rem(my + s, N_DEV)
                ccw = pltpu.make_async_remote_copy(
                    src_ref=x_ref if s == 0 else comm_ref.at[o_ccw],
                    dst_ref=comm_ref.at[o_ccw],
                    send_sem=ccw_send_sems.at[s],
                    recv_sem=ccw_recv_sems.at[s],
                    device_id=(left,),
                    device_id_type=pl.DeviceIdType.MESH,
                )
                ccw.start()
                ring_rdmas.append(ccw)

            if s == 0:
                blk = jnp.dot(x_ref[...], w_ref[...],
                              preferred_element_type=jnp.float32)
                out_ref[pl.ds(my * m_per, m_per), :] = jnp.maximum(blk, 0.0)
            else:
                gemm(lax.rem(my + (N_DEV - s), N_DEV))
                gemm(lax.rem(my + s, N_DEV))
            if s < n_cw:
                cw.wait_recv()
            if s < n_ccw:
                ccw.wait_recv()

        gemm(lax.rem(my + (N_DEV - n_cw), N_DEV))

        local_max = jnp.max(out_ref[...])
        amax_ref[my] = jnp.full((8, 128), local_max, jnp.float32)
        a_rdmas = []
        for d in range(1, N_DEV):
            tgt = lax.rem(my + d, N_DEV)
            r = pltpu.make_async_remote_copy(
                src_ref=amax_ref.at[my],
                dst_ref=amax_ref.at[my],
                send_sem=a_send_sems.at[d - 1],
                recv_sem=a_recv_sems.at[d - 1],
                device_id=(tgt,),
                device_id_type=pl.DeviceIdType.MESH,
            )
            r.start()
            a_rdmas.append(r)
        for r in a_rdmas:
            r.wait()

        gmax = jnp.max(amax_ref[...])
        scale = gmax / 127.0
        y = out_ref[...]
        q = jnp.clip(jnp.round(y / scale), 0.0, 127.0)
        out_ref[...] = q * scale

        for rdma in ring_rdmas:
            rdma.wait_send()

    return pl.pallas_call(
        body,
        out_shape=jax.ShapeDtypeStruct((N_DEV * m_per, n_per), jnp.float32),
        in_specs=[
            pl.BlockSpec(memory_space=pltpu.VMEM),
            pl.BlockSpec(memory_space=pltpu.VMEM),
        ],
        out_specs=pl.BlockSpec(memory_space=pltpu.VMEM),
        scratch_shapes=[
            pltpu.VMEM((N_DEV, m_per, k), jnp.bfloat16),
            pltpu.VMEM((N_DEV, 8, 128), jnp.float32),
            pltpu.SemaphoreType.DMA((n_cw,)),
            pltpu.SemaphoreType.DMA((n_cw,)),
            pltpu.SemaphoreType.DMA((n_ccw,)),
            pltpu.SemaphoreType.DMA((n_ccw,)),
            pltpu.SemaphoreType.DMA((N_DEV - 1,)),
            pltpu.SemaphoreType.DMA((N_DEV - 1,)),
        ],
        compiler_params=pltpu.CompilerParams(
            collective_id=0,
            vmem_limit_bytes=64 * 1024 * 1024,
        ),
    )(xb, wb)


# device time: 257607 ns/iter; 1.6663x vs baseline; 1.6140x over previous
import jax
import jax.numpy as jnp
from jax import lax
from jax.experimental import pallas as pl
from jax.experimental.pallas import tpu as pltpu

N_DEV = 32

PERM = [0, 3, 4, 7, 15, 12, 11, 8, 16, 19, 20, 23, 31, 28, 27, 24,
        25, 26, 29, 30, 22, 21, 18, 17, 9, 10, 13, 14, 6, 5, 2, 1]
PERM_INV = [0] * N_DEV
for _r, _p in enumerate(PERM):
    PERM_INV[_p] = _r


def kernel(x, w_mat):
    m_per, k = x.shape
    _, n_per = w_mat.shape

    xb = x.astype(jnp.bfloat16)
    wb = w_mat.astype(jnp.bfloat16)

    n_cw = N_DEV // 2
    n_ccw = N_DEV - 1 - n_cw

    ring = jnp.asarray(PERM, jnp.int32)
    r = jnp.asarray(PERM_INV, jnp.int32)[lax.axis_index("i")]
    sched = jnp.stack(
        [ring[(r - 1) % N_DEV], ring[(r + 1) % N_DEV]]
        + [ring[(r - s) % N_DEV] for s in range(n_cw + 1)]
        + [ring[(r + s) % N_DEV] for s in range(n_ccw + 1)]
    ).astype(jnp.int32)

    def body(x_ref, w_ref, sched_ref, out_ref, comm_ref, amax_ref,
             cw_send_sems, cw_recv_sems, ccw_send_sems, ccw_recv_sems,
             a_send_sems, a_recv_sems):
        my = lax.axis_index("i")
        left = sched_ref[0]
        right = sched_ref[1]
        cw_origin = lambda s: sched_ref[2 + s]
        ccw_origin = lambda s: sched_ref[2 + n_cw + 1 + s]

        barrier = pltpu.get_barrier_semaphore()
        pl.semaphore_signal(barrier, inc=1, device_id=(left,),
                            device_id_type=pl.DeviceIdType.MESH)
        pl.semaphore_signal(barrier, inc=1, device_id=(right,),
                            device_id_type=pl.DeviceIdType.MESH)
        pl.semaphore_wait(barrier, 2)

        def gemm(origin):
            blk = jnp.dot(comm_ref[origin], w_ref[...],
                          preferred_element_type=jnp.float32)
            out_ref[pl.ds(origin * m_per, m_per), :] = jnp.maximum(blk, 0.0)

        ring_rdmas = []
        for s in range(n_cw):
            if s < n_cw:
                o_cw = cw_origin(s)
                cw = pltpu.make_async_remote_copy(
                    src_ref=x_ref if s == 0 else comm_ref.at[o_cw],
                    dst_ref=comm_ref.at[o_cw],
                    send_sem=cw_send_sems.at[s],
                    recv_sem=cw_recv_sems.at[s],
                    device_id=(right,),
                    device_id_type=pl.DeviceIdType.MESH,
                )
                cw.start()
                ring_rdmas.append(cw)
            if s < n_ccw:
                o_ccw = ccw_origin(s)
                ccw = pltpu.make_async_remote_copy(
                    src_ref=x_ref if s == 0 else comm_ref.at[o_ccw],
                    dst_ref=comm_ref.at[o_ccw],
                    send_sem=ccw_send_sems.at[s],
                    recv_sem=ccw_recv_sems.at[s],
                    device_id=(left,),
                    device_id_type=pl.DeviceIdType.MESH,
                )
                ccw.start()
                ring_rdmas.append(ccw)

            if s == 0:
                blk = jnp.dot(x_ref[...], w_ref[...],
                              preferred_element_type=jnp.float32)
                out_ref[pl.ds(my * m_per, m_per), :] = jnp.maximum(blk, 0.0)
            else:
                gemm(cw_origin(s))
                gemm(ccw_origin(s))
            if s < n_cw:
                cw.wait_recv()
            if s < n_ccw:
                ccw.wait_recv()

        gemm(cw_origin(n_cw))

        local_max = jnp.max(out_ref[...])
        amax_ref[my] = jnp.full((8, 128), local_max, jnp.float32)
        a_rdmas = []
        for d in range(1, N_DEV):
            tgt = lax.rem(my + d, N_DEV)
            r = pltpu.make_async_remote_copy(
                src_ref=amax_ref.at[my],
                dst_ref=amax_ref.at[my],
                send_sem=a_send_sems.at[d - 1],
                recv_sem=a_recv_sems.at[d - 1],
                device_id=(tgt,),
                device_id_type=pl.DeviceIdType.MESH,
            )
            r.start()
            a_rdmas.append(r)
        for r in a_rdmas:
            r.wait()

        gmax = jnp.max(amax_ref[...])
        scale = gmax / 127.0
        y = out_ref[...]
        q = jnp.clip(jnp.round(y / scale), 0.0, 127.0)
        out_ref[...] = q * scale

        for rdma in ring_rdmas:
            rdma.wait_send()

    return pl.pallas_call(
        body,
        out_shape=jax.ShapeDtypeStruct((N_DEV * m_per, n_per), jnp.float32),
        in_specs=[
            pl.BlockSpec(memory_space=pltpu.VMEM),
            pl.BlockSpec(memory_space=pltpu.VMEM),
            pl.BlockSpec(memory_space=pltpu.SMEM),
        ],
        out_specs=pl.BlockSpec(memory_space=pltpu.VMEM),
        scratch_shapes=[
            pltpu.VMEM((N_DEV, m_per, k), jnp.bfloat16),
            pltpu.VMEM((N_DEV, 8, 128), jnp.float32),
            pltpu.SemaphoreType.DMA((n_cw,)),
            pltpu.SemaphoreType.DMA((n_cw,)),
            pltpu.SemaphoreType.DMA((n_ccw,)),
            pltpu.SemaphoreType.DMA((n_ccw,)),
            pltpu.SemaphoreType.DMA((N_DEV - 1,)),
            pltpu.SemaphoreType.DMA((N_DEV - 1,)),
        ],
        compiler_params=pltpu.CompilerParams(
            collective_id=0,
            vmem_limit_bytes=64 * 1024 * 1024,
        ),
    )(xb, wb, sched)


# device time: 228647 ns/iter; 1.8773x vs baseline; 1.1267x over previous
import numpy as np

import jax
import jax.numpy as jnp
from jax import lax
from jax.experimental import pallas as pl
from jax.experimental.pallas import tpu as pltpu

N_DEV = 32
N_CW = N_DEV // 2
N_CCW = N_DEV - 1 - N_CW

PERM = [0, 3, 4, 7, 15, 12, 11, 8, 16, 19, 20, 23, 31, 28, 27, 24,
        25, 26, 29, 30, 22, 21, 18, 17, 9, 10, 13, 14, 6, 5, 2, 1]
PERM_INV = [0] * N_DEV
for _r, _p in enumerate(PERM):
    PERM_INV[_p] = _r

_SCHED = np.zeros((N_DEV, 2 + (N_CW + 1) + (N_CCW + 1)), np.int32)
for _m in range(N_DEV):
    _r = PERM_INV[_m]
    _SCHED[_m] = (
        [PERM[(_r - 1) % N_DEV], PERM[(_r + 1) % N_DEV]]
        + [PERM[(_r - s) % N_DEV] for s in range(N_CW + 1)]
        + [PERM[(_r + s) % N_DEV] for s in range(N_CCW + 1)]
    )


def kernel(x, w_mat):
    m_per, k = x.shape
    _, n_per = w_mat.shape

    xb = x.astype(jnp.bfloat16)
    wb = w_mat.astype(jnp.bfloat16)

    sched = jnp.asarray(_SCHED)[lax.axis_index("i")]

    def body(x_ref, w_ref, sched_ref, out_ref, comm_ref, amax_ref,
             cw_send_sems, cw_recv_sems, ccw_send_sems, ccw_recv_sems,
             a_send_sems, a_recv_sems):
        my = lax.axis_index("i")
        left = sched_ref[0]
        right = sched_ref[1]
        cw_origin = lambda s: sched_ref[2 + s]
        ccw_origin = lambda s: sched_ref[2 + N_CW + 1 + s]

        barrier = pltpu.get_barrier_semaphore()
        pl.semaphore_signal(barrier, inc=1, device_id=(left,),
                            device_id_type=pl.DeviceIdType.MESH)
        pl.semaphore_signal(barrier, inc=1, device_id=(right,),
                            device_id_type=pl.DeviceIdType.MESH)
        pl.semaphore_wait(barrier, 2)

        def start_cw(s):
            r = pltpu.make_async_remote_copy(
                src_ref=x_ref if s == 0 else comm_ref.at[cw_origin(s)],
                dst_ref=comm_ref.at[cw_origin(s)],
                send_sem=cw_send_sems.at[s],
                recv_sem=cw_recv_sems.at[s],
                device_id=(right,),
                device_id_type=pl.DeviceIdType.MESH,
            )
            r.start()
            return r

        def start_ccw(s):
            r = pltpu.make_async_remote_copy(
                src_ref=x_ref if s == 0 else comm_ref.at[ccw_origin(s)],
                dst_ref=comm_ref.at[ccw_origin(s)],
                send_sem=ccw_send_sems.at[s],
                recv_sem=ccw_recv_sems.at[s],
                device_id=(left,),
                device_id_type=pl.DeviceIdType.MESH,
            )
            r.start()
            return r

        running_max = [jnp.float32(0.0)]

        def gemm(origin, src_ref):
            blk = jnp.maximum(
                jnp.dot(src_ref[...], w_ref[...],
                        preferred_element_type=jnp.float32),
                0.0,
            )
            out_ref[pl.ds(origin * m_per, m_per), :] = blk
            running_max[0] = jnp.maximum(running_max[0], jnp.max(blk))

        cw = start_cw(0)
        ccw = start_ccw(0)
        ring_rdmas = [cw, ccw]
        gemm(my, x_ref)

        for s in range(1, N_CW):
            cw.wait_recv()
            cw = start_cw(s)
            ring_rdmas.append(cw)
            gemm(cw_origin(s), comm_ref.at[cw_origin(s)])
            ccw.wait_recv()
            if s < N_CCW:
                ccw = start_ccw(s)
                ring_rdmas.append(ccw)
            gemm(ccw_origin(s), comm_ref.at[ccw_origin(s)])

        cw.wait_recv()
        gemm(cw_origin(N_CW), comm_ref.at[cw_origin(N_CW)])

        amax_ref[my] = jnp.full((8, 128), running_max[0], jnp.float32)
        a_rdmas = []
        for d in range(1, N_DEV):
            tgt = lax.rem(my + d, N_DEV)
            r = pltpu.make_async_remote_copy(
                src_ref=amax_ref.at[my],
                dst_ref=amax_ref.at[my],
                send_sem=a_send_sems.at[d - 1],
                recv_sem=a_recv_sems.at[d - 1],
                device_id=(tgt,),
                device_id_type=pl.DeviceIdType.MESH,
            )
            r.start()
            a_rdmas.append(r)
        for r in a_rdmas:
            r.wait()

        gmax = jnp.max(amax_ref[...])
        scale = gmax / 127.0
        y = out_ref[...]
        q = jnp.clip(jnp.round(y / scale), 0.0, 127.0)
        out_ref[...] = q * scale

        for rdma in ring_rdmas:
            rdma.wait_send()

    return pl.pallas_call(
        body,
        out_shape=jax.ShapeDtypeStruct((N_DEV * m_per, n_per), jnp.float32),
        in_specs=[
            pl.BlockSpec(memory_space=pltpu.VMEM),
            pl.BlockSpec(memory_space=pltpu.VMEM),
            pl.BlockSpec(memory_space=pltpu.SMEM),
        ],
        out_specs=pl.BlockSpec(memory_space=pltpu.VMEM),
        scratch_shapes=[
            pltpu.VMEM((N_DEV, m_per, k), jnp.bfloat16),
            pltpu.VMEM((N_DEV, 8, 128), jnp.float32),
            pltpu.SemaphoreType.DMA((N_CW,)),
            pltpu.SemaphoreType.DMA((N_CW,)),
            pltpu.SemaphoreType.DMA((N_CCW,)),
            pltpu.SemaphoreType.DMA((N_CCW,)),
            pltpu.SemaphoreType.DMA((N_DEV - 1,)),
            pltpu.SemaphoreType.DMA((N_DEV - 1,)),
        ],
        compiler_params=pltpu.CompilerParams(
            collective_id=0,
            vmem_limit_bytes=64 * 1024 * 1024,
        ),
    )(xb, wb, sched)


# device time: 204919 ns/iter; 2.0947x vs baseline; 1.1158x over previous
import numpy as np

import jax
import jax.numpy as jnp
from jax import lax
from jax.experimental import pallas as pl
from jax.experimental.pallas import tpu as pltpu

N_DEV = 32
N_CW = N_DEV // 2
N_CCW = N_DEV - 1 - N_CW

PERM = [0, 3, 4, 7, 15, 12, 11, 8, 16, 19, 20, 23, 31, 28, 27, 24,
        25, 26, 29, 30, 22, 21, 18, 17, 9, 10, 13, 14, 6, 5, 2, 1]
PERM_INV = [0] * N_DEV
for _r, _p in enumerate(PERM):
    PERM_INV[_p] = _r

_SCHED = np.zeros((N_DEV, 2 + (N_CW + 1) + (N_CCW + 1)), np.int32)
for _m in range(N_DEV):
    _r = PERM_INV[_m]
    _SCHED[_m] = (
        [PERM[(_r - 1) % N_DEV], PERM[(_r + 1) % N_DEV]]
        + [PERM[(_r - s) % N_DEV] for s in range(N_CW + 1)]
        + [PERM[(_r + s) % N_DEV] for s in range(N_CCW + 1)]
    )


def kernel(x, w_mat):
    m_per, k = x.shape
    _, n_per = w_mat.shape

    xb = x.astype(jnp.bfloat16)
    wb = w_mat.astype(jnp.bfloat16)

    hm = m_per // 2

    sched = jnp.asarray(_SCHED)[lax.axis_index("i")]

    def body(x_ref, w_ref, sched_ref, out_ref, comm_ref, amax_ref,
             cw_send_sems, cw_recv_sems, ccw_send_sems, ccw_recv_sems,
             a_send_sems, a_recv_sems):
        my = lax.axis_index("i")
        left = sched_ref[0]
        right = sched_ref[1]
        cw_origin = lambda s: sched_ref[2 + s]
        ccw_origin = lambda s: sched_ref[2 + N_CW + 1 + s]

        barrier = pltpu.get_barrier_semaphore()
        pl.semaphore_signal(barrier, inc=1, device_id=(left,),
                            device_id_type=pl.DeviceIdType.MESH)
        pl.semaphore_signal(barrier, inc=1, device_id=(right,),
                            device_id_type=pl.DeviceIdType.MESH)
        pl.semaphore_wait(barrier, 2)

        def start_cw(s, h):
            r = pltpu.make_async_remote_copy(
                src_ref=(x_ref.at[pl.ds(h * hm, hm)] if s == 0
                         else comm_ref.at[cw_origin(s), h]),
                dst_ref=comm_ref.at[cw_origin(s), h],
                send_sem=cw_send_sems.at[s, h],
                recv_sem=cw_recv_sems.at[s, h],
                device_id=(right,),
                device_id_type=pl.DeviceIdType.MESH,
            )
            r.start()
            return r

        def start_ccw(s, h):
            r = pltpu.make_async_remote_copy(
                src_ref=(x_ref.at[pl.ds(h * hm, hm)] if s == 0
                         else comm_ref.at[ccw_origin(s), h]),
                dst_ref=comm_ref.at[ccw_origin(s), h],
                send_sem=ccw_send_sems.at[s, h],
                recv_sem=ccw_recv_sems.at[s, h],
                device_id=(left,),
                device_id_type=pl.DeviceIdType.MESH,
            )
            r.start()
            return r

        running_max = [jnp.float32(0.0)]

        def gemm_half(origin, h):
            blk = jnp.maximum(
                jnp.dot(comm_ref[origin, h], w_ref[...],
                        preferred_element_type=jnp.float32),
                0.0,
            )
            out_ref[pl.ds(origin * m_per + h * hm, hm), :] = blk
            running_max[0] = jnp.maximum(running_max[0], jnp.max(blk))

        cw = [start_cw(0, 0), start_cw(0, 1)]
        ccw = [start_ccw(0, 0), start_ccw(0, 1)]
        ring_rdmas = cw + ccw

        blk = jnp.maximum(
            jnp.dot(x_ref[...], w_ref[...],
                    preferred_element_type=jnp.float32),
            0.0,
        )
        out_ref[pl.ds(my * m_per, m_per), :] = blk
        running_max[0] = jnp.max(blk)

        for s in range(1, N_CW):
            for h in range(2):
                cw[h].wait_recv()
                cw[h] = start_cw(s, h)
                ring_rdmas.append(cw[h])
                gemm_half(cw_origin(s), h)
            for h in range(2):
                ccw[h].wait_recv()
                if s < N_CCW:
                    ccw[h] = start_ccw(s, h)
                    ring_rdmas.append(ccw[h])
                gemm_half(ccw_origin(s), h)

        for h in range(2):
            cw[h].wait_recv()
            gemm_half(cw_origin(N_CW), h)

        amax_ref[my] = jnp.full((8, 128), running_max[0], jnp.float32)
        a_rdmas = []
        for d in range(1, N_DEV):
            tgt = lax.rem(my + d, N_DEV)
            r = pltpu.make_async_remote_copy(
                src_ref=amax_ref.at[my],
                dst_ref=amax_ref.at[my],
                send_sem=a_send_sems.at[d - 1],
                recv_sem=a_recv_sems.at[d - 1],
                device_id=(tgt,),
                device_id_type=pl.DeviceIdType.MESH,
            )
            r.start()
            a_rdmas.append(r)
        for r in a_rdmas:
            r.wait()

        gmax = jnp.max(amax_ref[...])
        scale = gmax / 127.0
        y = out_ref[...]
        q = jnp.clip(jnp.round(y / scale), 0.0, 127.0)
        out_ref[...] = q * scale

        for rdma in ring_rdmas:
            rdma.wait_send()

    return pl.pallas_call(
        body,
        out_shape=jax.ShapeDtypeStruct((N_DEV * m_per, n_per), jnp.float32),
        in_specs=[
            pl.BlockSpec(memory_space=pltpu.VMEM),
            pl.BlockSpec(memory_space=pltpu.VMEM),
            pl.BlockSpec(memory_space=pltpu.SMEM),
        ],
        out_specs=pl.BlockSpec(memory_space=pltpu.VMEM),
        scratch_shapes=[
            pltpu.VMEM((N_DEV, 2, hm, k), jnp.bfloat16),
            pltpu.VMEM((N_DEV, 8, 128), jnp.float32),
            pltpu.SemaphoreType.DMA((N_CW, 2)),
            pltpu.SemaphoreType.DMA((N_CW, 2)),
            pltpu.SemaphoreType.DMA((N_CCW, 2)),
            pltpu.SemaphoreType.DMA((N_CCW, 2)),
            pltpu.SemaphoreType.DMA((N_DEV - 1,)),
            pltpu.SemaphoreType.DMA((N_DEV - 1,)),
        ],
        compiler_params=pltpu.CompilerParams(
            collective_id=0,
            vmem_limit_bytes=64 * 1024 * 1024,
        ),
    )(xb, wb, sched)


# device time: 204042 ns/iter; 2.1037x vs baseline; 1.0043x over previous
import numpy as np

import jax
import jax.numpy as jnp
from jax import lax
from jax.experimental import pallas as pl
from jax.experimental.pallas import tpu as pltpu

N_DEV = 32
N_CW = N_DEV // 2
N_CCW = N_DEV - 1 - N_CW

PERM = [0, 3, 4, 7, 15, 12, 11, 8, 16, 19, 20, 23, 31, 28, 27, 24,
        25, 26, 29, 30, 22, 21, 18, 17, 9, 10, 13, 14, 6, 5, 2, 1]
PERM_INV = [0] * N_DEV
for _r, _p in enumerate(PERM):
    PERM_INV[_p] = _r

_SCHED = np.zeros((N_DEV, 2 + (N_CW + 1) + (N_CCW + 1)), np.int32)
for _m in range(N_DEV):
    _r = PERM_INV[_m]
    _SCHED[_m] = (
        [PERM[(_r - 1) % N_DEV], PERM[(_r + 1) % N_DEV]]
        + [PERM[(_r - s) % N_DEV] for s in range(N_CW + 1)]
        + [PERM[(_r + s) % N_DEV] for s in range(N_CCW + 1)]
    )


def kernel(x, w_mat):
    m_per, k = x.shape
    _, n_per = w_mat.shape

    xb = x.astype(jnp.bfloat16)
    wb = w_mat.astype(jnp.bfloat16)

    hm = m_per // 2

    sched = jnp.asarray(_SCHED)[lax.axis_index("i")]

    def body(x_ref, w_ref, sched_ref, out_ref, comm_ref, amax_ref,
             cw_send_sems, cw_recv_sems, ccw_send_sems, ccw_recv_sems,
             a_send_sems, a_recv_sems):
        my = lax.axis_index("i")
        left = sched_ref[0]
        right = sched_ref[1]
        cw_origin = lambda s: sched_ref[2 + s]
        ccw_origin = lambda s: sched_ref[2 + N_CW + 1 + s]

        barrier = pltpu.get_barrier_semaphore()
        pl.semaphore_signal(barrier, inc=1, device_id=(left,),
                            device_id_type=pl.DeviceIdType.MESH)
        pl.semaphore_signal(barrier, inc=1, device_id=(right,),
                            device_id_type=pl.DeviceIdType.MESH)
        pl.semaphore_wait(barrier, 2)

        def start_cw(s, h):
            r = pltpu.make_async_remote_copy(
                src_ref=(x_ref.at[pl.ds(h * hm, hm)] if s == 0
                         else comm_ref.at[cw_origin(s), h]),
                dst_ref=comm_ref.at[cw_origin(s), h],
                send_sem=cw_send_sems.at[s, h],
                recv_sem=cw_recv_sems.at[s, h],
                device_id=(right,),
                device_id_type=pl.DeviceIdType.MESH,
            )
            r.start()
            return r

        def start_ccw(s, h):
            r = pltpu.make_async_remote_copy(
                src_ref=(x_ref.at[pl.ds(h * hm, hm)] if s == 0
                         else comm_ref.at[ccw_origin(s), h]),
                dst_ref=comm_ref.at[ccw_origin(s), h],
                send_sem=ccw_send_sems.at[s, h],
                recv_sem=ccw_recv_sems.at[s, h],
                device_id=(left,),
                device_id_type=pl.DeviceIdType.MESH,
            )
            r.start()
            return r

        running_max = [jnp.float32(0.0)]

        def gemm_half(origin, h):
            blk = jnp.maximum(
                jnp.dot(comm_ref[origin, h], w_ref[...],
                        preferred_element_type=jnp.float32),
                0.0,
            )
            out_ref[pl.ds(origin * m_per + h * hm, hm), :] = blk
            running_max[0] = jnp.maximum(running_max[0], jnp.max(blk))

        cw = [start_cw(0, 0), start_cw(0, 1)]
        ccw = [start_ccw(0, 0), start_ccw(0, 1)]
        ring_rdmas = cw + ccw

        blk = jnp.maximum(
            jnp.dot(x_ref[...], w_ref[...],
                    preferred_element_type=jnp.float32),
            0.0,
        )
        out_ref[pl.ds(my * m_per, m_per), :] = blk
        running_max[0] = jnp.max(blk)

        for s in range(1, N_CW - 1):
            for h in range(2):
                cw[h].wait_recv()
                cw[h] = start_cw(s, h)
                ring_rdmas.append(cw[h])
                gemm_half(cw_origin(s), h)
            for h in range(2):
                ccw[h].wait_recv()
                ccw[h] = start_ccw(s, h)
                ring_rdmas.append(ccw[h])
                gemm_half(ccw_origin(s), h)

        s = N_CW - 1
        cw[0].wait_recv()
        cw[0] = start_cw(s, 0)
        ccw[1].wait_recv()
        ccw[1] = start_ccw(s, 1)
        ring_rdmas += [cw[0], ccw[1]]
        gemm_half(cw_origin(s), 0)
        cw[1].wait_recv()
        gemm_half(cw_origin(s), 1)
        ccw[0].wait_recv()
        gemm_half(ccw_origin(s), 0)
        gemm_half(ccw_origin(s), 1)

        cw[0].wait_recv()
        gemm_half(cw_origin(N_CW), 0)
        ccw[1].wait_recv()
        gemm_half(cw_origin(N_CW), 1)

        amax_ref[my] = jnp.full((8, 128), running_max[0], jnp.float32)
        a_rdmas = []
        for d in range(1, N_DEV):
            tgt = lax.rem(my + d, N_DEV)
            r = pltpu.make_async_remote_copy(
                src_ref=amax_ref.at[my],
                dst_ref=amax_ref.at[my],
                send_sem=a_send_sems.at[d - 1],
                recv_sem=a_recv_sems.at[d - 1],
                device_id=(tgt,),
                device_id_type=pl.DeviceIdType.MESH,
            )
            r.start()
            a_rdmas.append(r)
        for r in a_rdmas:
            r.wait()

        gmax = jnp.max(amax_ref[...])
        scale = gmax / 127.0
        y = out_ref[...]
        q = jnp.clip(jnp.round(y / scale), 0.0, 127.0)
        out_ref[...] = q * scale

        for rdma in ring_rdmas:
            rdma.wait_send()

    return pl.pallas_call(
        body,
        out_shape=jax.ShapeDtypeStruct((N_DEV * m_per, n_per), jnp.float32),
        in_specs=[
            pl.BlockSpec(memory_space=pltpu.VMEM),
            pl.BlockSpec(memory_space=pltpu.VMEM),
            pl.BlockSpec(memory_space=pltpu.SMEM),
        ],
        out_specs=pl.BlockSpec(memory_space=pltpu.VMEM),
        scratch_shapes=[
            pltpu.VMEM((N_DEV, 2, hm, k), jnp.bfloat16),
            pltpu.VMEM((N_DEV, 8, 128), jnp.float32),
            pltpu.SemaphoreType.DMA((N_CW, 2)),
            pltpu.SemaphoreType.DMA((N_CW, 2)),
            pltpu.SemaphoreType.DMA((N_CW, 2)),
            pltpu.SemaphoreType.DMA((N_CW, 2)),
            pltpu.SemaphoreType.DMA((N_DEV - 1,)),
            pltpu.SemaphoreType.DMA((N_DEV - 1,)),
        ],
        compiler_params=pltpu.CompilerParams(
            collective_id=0,
            vmem_limit_bytes=64 * 1024 * 1024,
        ),
    )(xb, wb, sched)
